# baseline (speedup 1.0000x reference)
import functools

import jax
import jax.numpy as jnp
from jax import lax
from jax.experimental import pallas as pl
from jax.experimental.pallas import tpu as pltpu
from jax.experimental.pallas import tpu_sc as plsc

N = 10000
D = 128
E = 320000
G = 256

NC, NS = 2, 16
NW = NC * NS
CH = 128
CPW = 80
EPW = CPW * CH
E_PAD = EPW * NW
PH = 8
NP = CPW // PH

ACC = 10240
RPT = ACC // NS

RCH = 2000

_mesh = plsc.VectorSubcoreMesh(
    core_axis_name="c", subcore_axis_name="s", num_cores=NC, num_subcores=NS)


@functools.partial(
    pl.kernel,
    out_type=jax.ShapeDtypeStruct((NC, ACC, D), jnp.float32),
    mesh=_mesh,
    scratch_types=[
        pltpu.VMEM((PH, CH), jnp.int32),
        pltpu.VMEM((PH, CH), jnp.int32),
        pltpu.VMEM((CH, D), jnp.float32),
        pltpu.VMEM((16, D), jnp.float32),
        pltpu.VMEM_SHARED((ACC, D), jnp.float32),
        pltpu.SemaphoreType.DMA,
    ],
)
def _deg_sc(dst_hbm, out_hbm, dr0, dr1, ones_v, zeros_v, acc, isem):
    dst_ring = (dr0, dr1)
    c = lax.axis_index("c")
    s = lax.axis_index("s")
    wid = c * NS + s

    def idx_start(p, seg):
        pltpu.make_async_copy(dst_hbm.at[wid, pl.ds(p * PH, PH)],
                              dst_ring[seg], isem).start()

    def idx_wait(seg):
        pltpu.make_async_copy(dst_hbm.at[wid, pl.ds(0, PH)],
                              dst_ring[seg], isem).wait()

    idx_start(0, 0)
    for i in range(CH):
        for j in range(D // 16):
            ones_v[i, pl.ds(j * 16, 16)] = jnp.ones((16,), jnp.float32)
    for i in range(16):
        for j in range(D // 16):
            zeros_v[i, pl.ds(j * 16, 16)] = jnp.zeros((16,), jnp.float32)

    def zbody(i, _):
        pltpu.sync_copy(zeros_v, acc.at[pl.ds(s * RPT + i * 16, 16)])
        return 0
    lax.fori_loop(0, RPT // 16, zbody, 0)
    plsc.subcore_barrier()

    def phase(p, seg):
        idx_wait(seg)

        @pl.when(p + 1 < NP)
        def _():
            idx_start(p + 1, 1 - seg)

        for k in range(PH):
            pltpu.sync_copy(ones_v, acc.at[dst_ring[seg].at[k]], add=True)

    def gbody(sp, _):
        phase(2 * sp, 0)
        phase(2 * sp + 1, 1)
        return 0
    lax.fori_loop(0, NP // 2, gbody, 0)
    plsc.subcore_barrier()
    pltpu.sync_copy(acc.at[pl.ds(s * RPT, RPT)],
                    out_hbm.at[c, pl.ds(s * RPT, RPT)])


@functools.partial(
    pl.kernel,
    out_type=jax.ShapeDtypeStruct((NC, ACC, D), jnp.float32),
    mesh=_mesh,
    scratch_types=[
        pltpu.VMEM((PH, CH), jnp.int32),
        pltpu.VMEM((PH, CH), jnp.int32),
        pltpu.VMEM((PH, CH), jnp.int32),
        pltpu.VMEM((PH, CH), jnp.int32),
        pltpu.VMEM((CH, D), jnp.float32),
        pltpu.VMEM((CH, D), jnp.float32),
        pltpu.VMEM((16, D), jnp.float32),
        pltpu.VMEM_SHARED((ACC, D), jnp.float32),
        pltpu.SemaphoreType.DMA,
        pltpu.SemaphoreType.DMA,
        pltpu.SemaphoreType.DMA,
    ],
)
def _agg_sc(y_hbm, src_hbm, dst_hbm, out_hbm, sr0, sr1, dr0, dr1, buf0, buf1,
            zeros_v, acc, isem, gs0, gs1):
    gsems = (gs0, gs1)
    bufs = (buf0, buf1)
    src_ring = (sr0, sr1)
    dst_ring = (dr0, dr1)
    c = lax.axis_index("c")
    s = lax.axis_index("s")
    wid = c * NS + s

    def idx_start(p, seg):
        pltpu.make_async_copy(src_hbm.at[wid, pl.ds(p * PH, PH)],
                              src_ring[seg], isem).start()
        pltpu.make_async_copy(dst_hbm.at[wid, pl.ds(p * PH, PH)],
                              dst_ring[seg], isem).start()

    def idx_wait(seg):
        pltpu.make_async_copy(src_hbm.at[wid, pl.ds(0, PH)],
                              src_ring[seg], isem).wait()
        pltpu.make_async_copy(dst_hbm.at[wid, pl.ds(0, PH)],
                              dst_ring[seg], isem).wait()

    idx_start(0, 0)
    for i in range(16):
        for j in range(D // 16):
            zeros_v[i, pl.ds(j * 16, 16)] = jnp.zeros((16,), jnp.float32)

    def zbody(i, _):
        pltpu.sync_copy(zeros_v, acc.at[pl.ds(s * RPT + i * 16, 16)])
        return 0
    lax.fori_loop(0, RPT // 16, zbody, 0)
    plsc.subcore_barrier()

    def gather_start(seg, k, b):
        pltpu.make_async_copy(y_hbm.at[src_ring[seg].at[k]], bufs[b],
                              gsems[b]).start()

    def gather_wait(seg, b):
        pltpu.make_async_copy(y_hbm.at[src_ring[seg].at[0]], bufs[b],
                              gsems[b]).wait()

    def phase(p, seg):
        idx_wait(seg)

        @pl.when(p + 1 < NP)
        def _():
            idx_start(p + 1, 1 - seg)

        gather_start(seg, 0, 0)
        for k in range(PH):
            if k + 1 < PH:
                gather_start(seg, k + 1, (k + 1) % 2)
            gather_wait(seg, k % 2)
            pltpu.sync_copy(bufs[k % 2], acc.at[dst_ring[seg].at[k]],
                            add=True)

    def gbody(sp, _):
        phase(2 * sp, 0)
        phase(2 * sp + 1, 1)
        return 0
    lax.fori_loop(0, NP // 2, gbody, 0)
    plsc.subcore_barrier()
    pltpu.sync_copy(acc.at[pl.ds(s * RPT, RPT)],
                    out_hbm.at[c, pl.ds(s * RPT, RPT)])


def _dinv_body(d0, d1, o):
    o[...] = lax.rsqrt(d0[...] + d1[...] + 1.0)


_dinv_tc = pl.pallas_call(
    _dinv_body,
    out_shape=jax.ShapeDtypeStruct((N, 1), jnp.float32),
    grid=(N // RCH,),
    in_specs=[pl.BlockSpec((RCH, 1), lambda i: (i, 0)),
              pl.BlockSpec((RCH, 1), lambda i: (i, 0))],
    out_specs=pl.BlockSpec((RCH, 1), lambda i: (i, 0)),
)


def _mm_scale_body(x, w, dinv, y):
    y[...] = dinv[...] * jnp.dot(x[...], w[...],
                                 preferred_element_type=jnp.float32)


_mm_scale_tc = pl.pallas_call(
    _mm_scale_body,
    out_shape=jax.ShapeDtypeStruct((N, D), jnp.float32),
    grid=(N // RCH,),
    in_specs=[pl.BlockSpec((RCH, D), lambda i: (i, 0)),
              pl.BlockSpec((D, D), lambda i: (0, 0)),
              pl.BlockSpec((RCH, 1), lambda i: (i, 0))],
    out_specs=pl.BlockSpec((RCH, D), lambda i: (i, 0)),
)


def _comb_body(a0, a1, yp, dinv, b, w, yn):
    h = jnp.maximum(dinv[...] * (a0[...] + a1[...] + yp[...]) + b[...], 0.0)
    yn[...] = dinv[...] * jnp.dot(h, w[...], preferred_element_type=jnp.float32)


_comb_tc = pl.pallas_call(
    _comb_body,
    out_shape=jax.ShapeDtypeStruct((N, D), jnp.float32),
    grid=(N // RCH,),
    in_specs=[pl.BlockSpec((RCH, D), lambda i: (i, 0)),
              pl.BlockSpec((RCH, D), lambda i: (i, 0)),
              pl.BlockSpec((RCH, D), lambda i: (i, 0)),
              pl.BlockSpec((RCH, 1), lambda i: (i, 0)),
              pl.BlockSpec((1, D), lambda i: (0, 0)),
              pl.BlockSpec((D, D), lambda i: (0, 0))],
    out_specs=pl.BlockSpec((RCH, D), lambda i: (i, 0)),
)


def _pool_body(a0, a1, yp, dinv, b, batch, out, sums, cnts):
    i = pl.program_id(0)

    @pl.when(i == 0)
    def _():
        sums[...] = jnp.zeros_like(sums)
        cnts[...] = jnp.zeros_like(cnts)

    h = jnp.maximum(dinv[...] * (a0[...] + a1[...] + yp[...]) + b[...], 0.0)
    bt = batch[...].reshape(1, RCH)
    gids = lax.broadcasted_iota(jnp.int32, (G, RCH), 0)
    onehot_t = (gids == bt).astype(jnp.float32)
    sums[...] += jnp.dot(onehot_t, h, preferred_element_type=jnp.float32)
    cnts[...] += jnp.dot(onehot_t, jnp.ones((RCH, D), jnp.float32),
                         preferred_element_type=jnp.float32)

    @pl.when(i == pl.num_programs(0) - 1)
    def _():
        out[...] = sums[...] / jnp.maximum(cnts[...], 1.0)


_pool_tc = pl.pallas_call(
    _pool_body,
    out_shape=jax.ShapeDtypeStruct((G, D), jnp.float32),
    grid=(N // RCH,),
    in_specs=[pl.BlockSpec((RCH, D), lambda i: (i, 0)),
              pl.BlockSpec((RCH, D), lambda i: (i, 0)),
              pl.BlockSpec((RCH, D), lambda i: (i, 0)),
              pl.BlockSpec((RCH, 1), lambda i: (i, 0)),
              pl.BlockSpec((1, D), lambda i: (0, 0)),
              pl.BlockSpec((1, 1, RCH), lambda i: (i, 0, 0))],
    out_specs=pl.BlockSpec((G, D), lambda i: (0, 0)),
    scratch_shapes=[pltpu.VMEM((G, D), jnp.float32),
                    pltpu.VMEM((G, D), jnp.float32)],
)


@jax.jit
def kernel(x, edge_index, batch, W1, b1, W2, b2, W3, b3, W4, b4):
    src = edge_index[0].astype(jnp.int32)
    dst = edge_index[1].astype(jnp.int32)
    srcp = jnp.concatenate([src, jnp.zeros((E_PAD - E,), jnp.int32)])
    dstp = jnp.concatenate([dst, jnp.full((E_PAD - E,), N, jnp.int32)])
    srcp = srcp.reshape(NW, CPW, CH)
    dstp = dstp.reshape(NW, CPW, CH)
    batch2 = batch.astype(jnp.int32).reshape(N // RCH, 1, RCH)

    degp = _deg_sc(dstp)
    d0 = degp[0, :N, 0:1]
    d1 = degp[1, :N, 0:1]
    dinv = _dinv_tc(d0, d1)

    b1r = b1.reshape(1, D)
    b2r = b2.reshape(1, D)
    b3r = b3.reshape(1, D)
    b4r = b4.reshape(1, D)

    y = _mm_scale_tc(x, W1, dinv)
    for (br, Wn) in ((b1r, W2), (b2r, W3), (b3r, W4)):
        ag = _agg_sc(y, srcp, dstp)
        y = _comb_tc(ag[0, :N], ag[1, :N], y, dinv, br, Wn)
    ag = _agg_sc(y, srcp, dstp)
    out = _pool_tc(ag[0, :N], ag[1, :N], y, dinv, b4r, batch2)
    return out


# speedup vs baseline: 3.1239x; 3.1239x over previous
import functools

import jax
import jax.numpy as jnp
from jax import lax
from jax.experimental import pallas as pl
from jax.experimental.pallas import tpu as pltpu
from jax.experimental.pallas import tpu_sc as plsc

N = 10000
D = 128
E = 320000
G = 256

NC, NS = 2, 16
NW = NC * NS
CH = 128
CPW = 80
EPW = CPW * CH
E_PAD = EPW * NW
PH = 8
NP = CPW // PH

ACC = 10240
RPT = ACC // NS

RCH = 2000

_mesh = plsc.VectorSubcoreMesh(
    core_axis_name="c", subcore_axis_name="s", num_cores=NC, num_subcores=NS)


@functools.partial(
    pl.kernel,
    out_type=jax.ShapeDtypeStruct((NC, ACC, D), jnp.float32),
    mesh=_mesh,
    scratch_types=[
        pltpu.VMEM((PH, CH), jnp.int32),
        pltpu.VMEM((PH, CH), jnp.int32),
        pltpu.VMEM((CH, D), jnp.float32),
        pltpu.VMEM((16, D), jnp.float32),
        pltpu.VMEM_SHARED((ACC, D), jnp.float32),
        pltpu.SemaphoreType.DMA,
    ],
)
def _deg_sc(dst_hbm, out_hbm, dr0, dr1, ones_v, zeros_v, acc, isem):
    dst_ring = (dr0, dr1)
    c = lax.axis_index("c")
    s = lax.axis_index("s")
    wid = c * NS + s

    def idx_start(p, seg):
        pltpu.make_async_copy(dst_hbm.at[wid, pl.ds(p * PH, PH)],
                              dst_ring[seg], isem).start()

    def idx_wait(seg):
        pltpu.make_async_copy(dst_hbm.at[wid, pl.ds(0, PH)],
                              dst_ring[seg], isem).wait()

    idx_start(0, 0)
    for i in range(CH):
        for j in range(D // 16):
            ones_v[i, pl.ds(j * 16, 16)] = jnp.ones((16,), jnp.float32)
    for i in range(16):
        for j in range(D // 16):
            zeros_v[i, pl.ds(j * 16, 16)] = jnp.zeros((16,), jnp.float32)

    def zbody(i, _):
        pltpu.sync_copy(zeros_v, acc.at[pl.ds(s * RPT + i * 16, 16)])
        return 0
    lax.fori_loop(0, RPT // 16, zbody, 0)
    plsc.subcore_barrier()

    def phase(p, seg):
        idx_wait(seg)

        @pl.when(p + 1 < NP)
        def _():
            idx_start(p + 1, 1 - seg)

        for k in range(PH):
            pltpu.sync_copy(ones_v, acc.at[dst_ring[seg].at[k]], add=True)

    def gbody(sp, _):
        phase(2 * sp, 0)
        phase(2 * sp + 1, 1)
        return 0
    lax.fori_loop(0, NP // 2, gbody, 0)
    plsc.subcore_barrier()
    pltpu.sync_copy(acc.at[pl.ds(s * RPT, RPT)],
                    out_hbm.at[c, pl.ds(s * RPT, RPT)])


@functools.partial(
    pl.kernel,
    out_type=jax.ShapeDtypeStruct((NC, ACC, D), jnp.float32),
    mesh=_mesh,
    scratch_types=[
        pltpu.VMEM((PH, CH), jnp.int32),
        pltpu.VMEM((PH, CH), jnp.int32),
        pltpu.VMEM((PH, CH), jnp.int32),
        pltpu.VMEM((PH, CH), jnp.int32),
        pltpu.VMEM((CH, D), jnp.float32),
        pltpu.VMEM((CH, D), jnp.float32),
        pltpu.VMEM((16, D), jnp.float32),
        pltpu.VMEM_SHARED((ACC, D), jnp.float32),
        pltpu.SemaphoreType.DMA,
        pltpu.SemaphoreType.DMA,
        pltpu.SemaphoreType.DMA,
    ],
)
def _agg_sc(y_hbm, src_hbm, dst_hbm, out_hbm, sr0, sr1, dr0, dr1, buf0, buf1,
            zeros_v, acc, isem, gs0, gs1):
    gsems = (gs0, gs1)
    bufs = (buf0, buf1)
    src_ring = (sr0, sr1)
    dst_ring = (dr0, dr1)
    c = lax.axis_index("c")
    s = lax.axis_index("s")
    wid = c * NS + s

    def idx_start(p, seg):
        pltpu.make_async_copy(src_hbm.at[wid, pl.ds(p * PH, PH)],
                              src_ring[seg], isem).start()
        pltpu.make_async_copy(dst_hbm.at[wid, pl.ds(p * PH, PH)],
                              dst_ring[seg], isem).start()

    def idx_wait(seg):
        pltpu.make_async_copy(src_hbm.at[wid, pl.ds(0, PH)],
                              src_ring[seg], isem).wait()
        pltpu.make_async_copy(dst_hbm.at[wid, pl.ds(0, PH)],
                              dst_ring[seg], isem).wait()

    idx_start(0, 0)
    for i in range(16):
        for j in range(D // 16):
            zeros_v[i, pl.ds(j * 16, 16)] = jnp.zeros((16,), jnp.float32)

    def zbody(i, _):
        pltpu.sync_copy(zeros_v, acc.at[pl.ds(s * RPT + i * 16, 16)])
        return 0
    lax.fori_loop(0, RPT // 16, zbody, 0)
    plsc.subcore_barrier()

    def gather_start(seg, k, b):
        pltpu.make_async_copy(y_hbm.at[src_ring[seg].at[k]], bufs[b],
                              gsems[b]).start()

    def gather_wait(seg, b):
        pltpu.make_async_copy(y_hbm.at[src_ring[seg].at[0]], bufs[b],
                              gsems[b]).wait()

    def phase(p, seg):
        idx_wait(seg)

        @pl.when(p + 1 < NP)
        def _():
            idx_start(p + 1, 1 - seg)

        gather_start(seg, 0, 0)
        for k in range(PH):
            if k + 1 < PH:
                gather_start(seg, k + 1, (k + 1) % 2)
            gather_wait(seg, k % 2)
            pltpu.sync_copy(bufs[k % 2], acc.at[dst_ring[seg].at[k]],
                            add=True)

    def gbody(sp, _):
        phase(2 * sp, 0)
        phase(2 * sp + 1, 1)
        return 0
    lax.fori_loop(0, NP // 2, gbody, 0)
    plsc.subcore_barrier()
    pltpu.sync_copy(acc.at[pl.ds(s * RPT, RPT)],
                    out_hbm.at[c, pl.ds(s * RPT, RPT)])


def _dinv_body(d0, d1, o):
    o[...] = lax.rsqrt(d0[...] + d1[...] + 1.0)


_dinv_tc = pl.pallas_call(
    _dinv_body,
    out_shape=jax.ShapeDtypeStruct((N, 1), jnp.float32),
    grid=(N // RCH,),
    in_specs=[pl.BlockSpec((RCH, 1), lambda i: (i, 0)),
              pl.BlockSpec((RCH, 1), lambda i: (i, 0))],
    out_specs=pl.BlockSpec((RCH, 1), lambda i: (i, 0)),
)


def _mm_scale_body(x, w, dinv, y):
    y[...] = dinv[...] * jnp.dot(x[...], w[...],
                                 preferred_element_type=jnp.float32)


_mm_scale_tc = pl.pallas_call(
    _mm_scale_body,
    out_shape=jax.ShapeDtypeStruct((N, D), jnp.float32),
    grid=(N // RCH,),
    in_specs=[pl.BlockSpec((RCH, D), lambda i: (i, 0)),
              pl.BlockSpec((D, D), lambda i: (0, 0)),
              pl.BlockSpec((RCH, 1), lambda i: (i, 0))],
    out_specs=pl.BlockSpec((RCH, D), lambda i: (i, 0)),
)


def _comb_body(a0, a1, yp, dinv, b, w, yn):
    h = jnp.maximum(dinv[...] * (a0[...] + a1[...] + yp[...]) + b[...], 0.0)
    yn[...] = dinv[...] * jnp.dot(h, w[...], preferred_element_type=jnp.float32)


_comb_tc = pl.pallas_call(
    _comb_body,
    out_shape=jax.ShapeDtypeStruct((N, D), jnp.float32),
    grid=(N // RCH,),
    in_specs=[pl.BlockSpec((RCH, D), lambda i: (i, 0)),
              pl.BlockSpec((RCH, D), lambda i: (i, 0)),
              pl.BlockSpec((RCH, D), lambda i: (i, 0)),
              pl.BlockSpec((RCH, 1), lambda i: (i, 0)),
              pl.BlockSpec((1, D), lambda i: (0, 0)),
              pl.BlockSpec((D, D), lambda i: (0, 0))],
    out_specs=pl.BlockSpec((RCH, D), lambda i: (i, 0)),
)


def _pool_body(a0, a1, yp, dinv, b, batch, out, sums, cnts):
    i = pl.program_id(0)

    @pl.when(i == 0)
    def _():
        sums[...] = jnp.zeros_like(sums)
        cnts[...] = jnp.zeros_like(cnts)

    h = jnp.maximum(dinv[...] * (a0[...] + a1[...] + yp[...]) + b[...], 0.0)
    bt = batch[...].reshape(1, RCH)
    gids = lax.broadcasted_iota(jnp.int32, (G, RCH), 0)
    onehot_t = (gids == bt).astype(jnp.float32)
    sums[...] += jnp.dot(onehot_t, h, preferred_element_type=jnp.float32)
    cnts[...] += jnp.dot(onehot_t, jnp.ones((RCH, D), jnp.float32),
                         preferred_element_type=jnp.float32)

    @pl.when(i == pl.num_programs(0) - 1)
    def _():
        out[...] = sums[...] / jnp.maximum(cnts[...], 1.0)


_pool_tc = pl.pallas_call(
    _pool_body,
    out_shape=jax.ShapeDtypeStruct((G, D), jnp.float32),
    grid=(N // RCH,),
    in_specs=[pl.BlockSpec((RCH, D), lambda i: (i, 0)),
              pl.BlockSpec((RCH, D), lambda i: (i, 0)),
              pl.BlockSpec((RCH, D), lambda i: (i, 0)),
              pl.BlockSpec((RCH, 1), lambda i: (i, 0)),
              pl.BlockSpec((1, D), lambda i: (0, 0)),
              pl.BlockSpec((1, 1, RCH), lambda i: (i, 0, 0))],
    out_specs=pl.BlockSpec((G, D), lambda i: (0, 0)),
    scratch_shapes=[pltpu.VMEM((G, D), jnp.float32),
                    pltpu.VMEM((G, D), jnp.float32)],
)


@jax.jit
def kernel(x, edge_index, batch, W1, b1, W2, b2, W3, b3, W4, b4):
    src = edge_index[0].astype(jnp.int32)
    dst = edge_index[1].astype(jnp.int32)
    npad = E_PAD - E
    pad_src = (jnp.arange(npad, dtype=jnp.int32) * 64) % N
    pad_dst = N + (jnp.arange(npad, dtype=jnp.int32) % (ACC - N))
    srcp = jnp.concatenate([src, pad_src])
    dstp = jnp.concatenate([dst, pad_dst])
    srcp = srcp.reshape(NW, CPW, CH)
    dstp = dstp.reshape(NW, CPW, CH)
    batch2 = batch.astype(jnp.int32).reshape(N // RCH, 1, RCH)

    degp = _deg_sc(dstp)
    d0 = degp[0, :N, 0:1]
    d1 = degp[1, :N, 0:1]
    dinv = _dinv_tc(d0, d1)

    b1r = b1.reshape(1, D)
    b2r = b2.reshape(1, D)
    b3r = b3.reshape(1, D)
    b4r = b4.reshape(1, D)

    y = _mm_scale_tc(x, W1, dinv)
    for (br, Wn) in ((b1r, W2), (b2r, W3), (b3r, W4)):
        ag = _agg_sc(y, srcp, dstp)
        y = _comb_tc(ag[0, :N], ag[1, :N], y, dinv, br, Wn)
    ag = _agg_sc(y, srcp, dstp)
    out = _pool_tc(ag[0, :N], ag[1, :N], y, dinv, b4r, batch2)
    return out


# speedup vs baseline: 3.2938x; 1.0544x over previous
import functools

import jax
import jax.numpy as jnp
from jax import lax
from jax.experimental import pallas as pl
from jax.experimental.pallas import tpu as pltpu
from jax.experimental.pallas import tpu_sc as plsc

N = 10000
D = 128
E = 320000
G = 256

NC, NS = 2, 16
NW = NC * NS
CH = 128
CPW = 80
EPW = CPW * CH
E_PAD = EPW * NW
PH = 8
NP = CPW // PH

ACC = 10240
RPT = ACC // NS

RCH = 2000

_mesh = plsc.VectorSubcoreMesh(
    core_axis_name="c", subcore_axis_name="s", num_cores=NC, num_subcores=NS)


@functools.partial(
    pl.kernel,
    out_type=jax.ShapeDtypeStruct((NC, ACC, D), jnp.float32),
    mesh=_mesh,
    scratch_types=[
        pltpu.VMEM((PH, CH), jnp.int32),
        pltpu.VMEM((PH, CH), jnp.int32),
        pltpu.VMEM((CH, D), jnp.float32),
        pltpu.VMEM((16, D), jnp.float32),
        pltpu.VMEM_SHARED((ACC, D), jnp.float32),
        pltpu.SemaphoreType.DMA,
    ],
)
def _deg_sc(dst_hbm, out_hbm, dr0, dr1, ones_v, zeros_v, acc, isem):
    dst_ring = (dr0, dr1)
    c = lax.axis_index("c")
    s = lax.axis_index("s")
    wid = c * NS + s

    def idx_start(p, seg):
        pltpu.make_async_copy(dst_hbm.at[wid, pl.ds(p * PH, PH)],
                              dst_ring[seg], isem).start()

    def idx_wait(seg):
        pltpu.make_async_copy(dst_hbm.at[wid, pl.ds(0, PH)],
                              dst_ring[seg], isem).wait()

    idx_start(0, 0)
    for i in range(CH):
        for j in range(D // 16):
            ones_v[i, pl.ds(j * 16, 16)] = jnp.ones((16,), jnp.float32)
    for i in range(16):
        for j in range(D // 16):
            zeros_v[i, pl.ds(j * 16, 16)] = jnp.zeros((16,), jnp.float32)

    def zbody(i, _):
        pltpu.sync_copy(zeros_v, acc.at[pl.ds(s * RPT + i * 16, 16)])
        return 0
    lax.fori_loop(0, RPT // 16, zbody, 0)
    plsc.subcore_barrier()

    def phase(p, seg):
        idx_wait(seg)

        @pl.when(p + 1 < NP)
        def _():
            idx_start(p + 1, 1 - seg)

        for k in range(PH):
            pltpu.sync_copy(ones_v, acc.at[dst_ring[seg].at[k]], add=True)

    def gbody(sp, _):
        phase(2 * sp, 0)
        phase(2 * sp + 1, 1)
        return 0
    lax.fori_loop(0, NP // 2, gbody, 0)
    plsc.subcore_barrier()
    pltpu.sync_copy(acc.at[pl.ds(s * RPT, RPT)],
                    out_hbm.at[c, pl.ds(s * RPT, RPT)])


@functools.partial(
    pl.kernel,
    out_type=jax.ShapeDtypeStruct((NC, ACC, D), jnp.float32),
    mesh=_mesh,
    scratch_types=[
        pltpu.VMEM((PH, CH), jnp.int32),
        pltpu.VMEM((PH, CH), jnp.int32),
        pltpu.VMEM((PH, CH), jnp.int32),
        pltpu.VMEM((PH, CH), jnp.int32),
        pltpu.VMEM((CH, D), jnp.float32),
        pltpu.VMEM((CH, D), jnp.float32),
        pltpu.VMEM((16, D), jnp.float32),
        pltpu.VMEM_SHARED((ACC, D), jnp.float32),
        pltpu.SemaphoreType.DMA,
        pltpu.SemaphoreType.DMA,
        pltpu.SemaphoreType.DMA,
        pltpu.SemaphoreType.DMA,
        pltpu.SemaphoreType.DMA,
    ],
)
def _agg_sc(y_hbm, src_hbm, dst_hbm, out_hbm, sr0, sr1, dr0, dr1, buf0, buf1,
            zeros_v, acc, isem, gs0, gs1, ss0, ss1):
    gsems = (gs0, gs1)
    ssems = (ss0, ss1)
    bufs = (buf0, buf1)
    src_ring = (sr0, sr1)
    dst_ring = (dr0, dr1)
    c = lax.axis_index("c")
    s = lax.axis_index("s")
    wid = c * NS + s

    def idx_start(p, seg):
        pltpu.make_async_copy(src_hbm.at[wid, pl.ds(p * PH, PH)],
                              src_ring[seg], isem).start()
        pltpu.make_async_copy(dst_hbm.at[wid, pl.ds(p * PH, PH)],
                              dst_ring[seg], isem).start()

    def idx_wait(seg):
        pltpu.make_async_copy(src_hbm.at[wid, pl.ds(0, PH)],
                              src_ring[seg], isem).wait()
        pltpu.make_async_copy(dst_hbm.at[wid, pl.ds(0, PH)],
                              dst_ring[seg], isem).wait()

    idx_start(0, 0)
    for i in range(16):
        for j in range(D // 16):
            zeros_v[i, pl.ds(j * 16, 16)] = jnp.zeros((16,), jnp.float32)

    def zbody(i, _):
        pltpu.sync_copy(zeros_v, acc.at[pl.ds(s * RPT + i * 16, 16)])
        return 0
    lax.fori_loop(0, RPT // 16, zbody, 0)
    plsc.subcore_barrier()

    def gather_start(seg, k, b):
        pltpu.make_async_copy(y_hbm.at[src_ring[seg].at[k]], bufs[b],
                              gsems[b]).start()

    def gather_wait(seg, b):
        pltpu.make_async_copy(y_hbm.at[src_ring[seg].at[0]], bufs[b],
                              gsems[b]).wait()

    def scatter_start(seg, k, b):
        pltpu.async_copy(bufs[b], acc.at[dst_ring[seg].at[k]], ssems[b],
                         add=True)

    def scatter_wait(seg, b):
        pltpu.make_async_copy(bufs[b], acc.at[dst_ring[seg].at[0]],
                              ssems[b]).wait()

    def phase(p, seg, lead=False):
        idx_wait(seg)

        @pl.when(p + 1 < NP)
        def _():
            idx_start(p + 1, 1 - seg)

        gather_start(seg, 0, 0)
        for k in range(PH):
            b = k % 2
            if not (lead and k == 0):
                scatter_wait(seg, 1 - b)
            if k + 1 < PH:
                gather_start(seg, k + 1, 1 - b)
            gather_wait(seg, b)
            scatter_start(seg, k, b)

    phase(0, 0, lead=True)
    phase(1, 1)

    def gbody(sp, _):
        phase(2 * sp + 2, 0)
        phase(2 * sp + 3, 1)
        return 0
    lax.fori_loop(0, (NP - 2) // 2, gbody, 0)
    scatter_wait(1, (PH - 1) % 2)
    plsc.subcore_barrier()
    pltpu.sync_copy(acc.at[pl.ds(s * RPT, RPT)],
                    out_hbm.at[c, pl.ds(s * RPT, RPT)])


def _dinv_body(d0, d1, o):
    o[...] = lax.rsqrt(d0[...] + d1[...] + 1.0)


_dinv_tc = pl.pallas_call(
    _dinv_body,
    out_shape=jax.ShapeDtypeStruct((N, 1), jnp.float32),
    grid=(N // RCH,),
    in_specs=[pl.BlockSpec((RCH, 1), lambda i: (i, 0)),
              pl.BlockSpec((RCH, 1), lambda i: (i, 0))],
    out_specs=pl.BlockSpec((RCH, 1), lambda i: (i, 0)),
)


def _mm_scale_body(x, w, dinv, y):
    y[...] = dinv[...] * jnp.dot(x[...], w[...],
                                 preferred_element_type=jnp.float32)


_mm_scale_tc = pl.pallas_call(
    _mm_scale_body,
    out_shape=jax.ShapeDtypeStruct((N, D), jnp.float32),
    grid=(N // RCH,),
    in_specs=[pl.BlockSpec((RCH, D), lambda i: (i, 0)),
              pl.BlockSpec((D, D), lambda i: (0, 0)),
              pl.BlockSpec((RCH, 1), lambda i: (i, 0))],
    out_specs=pl.BlockSpec((RCH, D), lambda i: (i, 0)),
)


def _comb_body(a0, a1, yp, dinv, b, w, yn):
    h = jnp.maximum(dinv[...] * (a0[...] + a1[...] + yp[...]) + b[...], 0.0)
    yn[...] = dinv[...] * jnp.dot(h, w[...], preferred_element_type=jnp.float32)


_comb_tc = pl.pallas_call(
    _comb_body,
    out_shape=jax.ShapeDtypeStruct((N, D), jnp.float32),
    grid=(N // RCH,),
    in_specs=[pl.BlockSpec((RCH, D), lambda i: (i, 0)),
              pl.BlockSpec((RCH, D), lambda i: (i, 0)),
              pl.BlockSpec((RCH, D), lambda i: (i, 0)),
              pl.BlockSpec((RCH, 1), lambda i: (i, 0)),
              pl.BlockSpec((1, D), lambda i: (0, 0)),
              pl.BlockSpec((D, D), lambda i: (0, 0))],
    out_specs=pl.BlockSpec((RCH, D), lambda i: (i, 0)),
)


def _pool_body(a0, a1, yp, dinv, b, batch, out, sums, cnts):
    i = pl.program_id(0)

    @pl.when(i == 0)
    def _():
        sums[...] = jnp.zeros_like(sums)
        cnts[...] = jnp.zeros_like(cnts)

    h = jnp.maximum(dinv[...] * (a0[...] + a1[...] + yp[...]) + b[...], 0.0)
    bt = batch[...].reshape(1, RCH)
    gids = lax.broadcasted_iota(jnp.int32, (G, RCH), 0)
    onehot_t = (gids == bt).astype(jnp.float32)
    sums[...] += jnp.dot(onehot_t, h, preferred_element_type=jnp.float32)
    cnts[...] += jnp.dot(onehot_t, jnp.ones((RCH, D), jnp.float32),
                         preferred_element_type=jnp.float32)

    @pl.when(i == pl.num_programs(0) - 1)
    def _():
        out[...] = sums[...] / jnp.maximum(cnts[...], 1.0)


_pool_tc = pl.pallas_call(
    _pool_body,
    out_shape=jax.ShapeDtypeStruct((G, D), jnp.float32),
    grid=(N // RCH,),
    in_specs=[pl.BlockSpec((RCH, D), lambda i: (i, 0)),
              pl.BlockSpec((RCH, D), lambda i: (i, 0)),
              pl.BlockSpec((RCH, D), lambda i: (i, 0)),
              pl.BlockSpec((RCH, 1), lambda i: (i, 0)),
              pl.BlockSpec((1, D), lambda i: (0, 0)),
              pl.BlockSpec((1, 1, RCH), lambda i: (i, 0, 0))],
    out_specs=pl.BlockSpec((G, D), lambda i: (0, 0)),
    scratch_shapes=[pltpu.VMEM((G, D), jnp.float32),
                    pltpu.VMEM((G, D), jnp.float32)],
)


@jax.jit
def kernel(x, edge_index, batch, W1, b1, W2, b2, W3, b3, W4, b4):
    src = edge_index[0].astype(jnp.int32)
    dst = edge_index[1].astype(jnp.int32)
    npad = E_PAD - E
    pad_src = (jnp.arange(npad, dtype=jnp.int32) * 64) % N
    pad_dst = N + (jnp.arange(npad, dtype=jnp.int32) % (ACC - N))
    srcp = jnp.concatenate([src, pad_src])
    dstp = jnp.concatenate([dst, pad_dst])
    srcp = srcp.reshape(NW, CPW, CH)
    dstp = dstp.reshape(NW, CPW, CH)
    batch2 = batch.astype(jnp.int32).reshape(N // RCH, 1, RCH)

    degp = _deg_sc(dstp)
    d0 = degp[0, :N, 0:1]
    d1 = degp[1, :N, 0:1]
    dinv = _dinv_tc(d0, d1)

    b1r = b1.reshape(1, D)
    b2r = b2.reshape(1, D)
    b3r = b3.reshape(1, D)
    b4r = b4.reshape(1, D)

    y = _mm_scale_tc(x, W1, dinv)
    for (br, Wn) in ((b1r, W2), (b2r, W3), (b3r, W4)):
        ag = _agg_sc(y, srcp, dstp)
        y = _comb_tc(ag[0, :N], ag[1, :N], y, dinv, br, Wn)
    ag = _agg_sc(y, srcp, dstp)
    out = _pool_tc(ag[0, :N], ag[1, :N], y, dinv, b4r, batch2)
    return out


# speedup vs baseline: 3.5493x; 1.0776x over previous
import functools

import jax
import jax.numpy as jnp
from jax import lax
from jax.experimental import pallas as pl
from jax.experimental.pallas import tpu as pltpu
from jax.experimental.pallas import tpu_sc as plsc

N = 10000
D = 128
E = 320000
G = 256

NC, NS = 2, 16
NW = NC * NS
CH = 128
CPW = 80
EPW = CPW * CH
E_PAD = EPW * NW
PH = 8
NP = CPW // PH

ACC = 10240
RPT = ACC // NS

RCH = 2000

_mesh = plsc.VectorSubcoreMesh(
    core_axis_name="c", subcore_axis_name="s", num_cores=NC, num_subcores=NS)


@functools.partial(
    pl.kernel,
    out_type=jax.ShapeDtypeStruct((NC, ACC, D), jnp.float32),
    mesh=_mesh,
    scratch_types=[
        pltpu.VMEM((PH, CH), jnp.int32),
        pltpu.VMEM((PH, CH), jnp.int32),
        pltpu.VMEM((CH, D), jnp.float32),
        pltpu.VMEM((16, D), jnp.float32),
        pltpu.VMEM_SHARED((ACC, D), jnp.float32),
        pltpu.SemaphoreType.DMA,
        pltpu.SemaphoreType.DMA,
    ],
)
def _deg_sc(dst_hbm, out_hbm, dr0, dr1, ones_v, zeros_v, acc, isem, ssem):
    dst_ring = (dr0, dr1)
    c = lax.axis_index("c")
    s = lax.axis_index("s")
    wid = c * NS + s

    def idx_start(p, seg):
        pltpu.make_async_copy(dst_hbm.at[wid, pl.ds(p * PH, PH)],
                              dst_ring[seg], isem).start()

    def idx_wait(seg):
        pltpu.make_async_copy(dst_hbm.at[wid, pl.ds(0, PH)],
                              dst_ring[seg], isem).wait()

    idx_start(0, 0)
    for i in range(CH):
        for j in range(D // 16):
            ones_v[i, pl.ds(j * 16, 16)] = jnp.ones((16,), jnp.float32)
    for i in range(16):
        for j in range(D // 16):
            zeros_v[i, pl.ds(j * 16, 16)] = jnp.zeros((16,), jnp.float32)

    def zbody(i, _):
        pltpu.make_async_copy(zeros_v, acc.at[pl.ds(s * RPT + i * 16, 16)],
                              ssem).start()
        return 0
    lax.fori_loop(0, RPT // 16, zbody, 0)

    def zdrain(i, _):
        pltpu.make_async_copy(zeros_v, acc.at[pl.ds(s * RPT, 16)],
                              ssem).wait()
        return 0
    lax.fori_loop(0, RPT // 16, zdrain, 0)
    plsc.subcore_barrier()

    def scatter_drain(seg):
        for k in range(PH):
            pltpu.make_async_copy(ones_v, acc.at[dst_ring[seg].at[0]],
                                  ssem).wait()

    def phase(p, seg, lead=False):
        idx_wait(seg)
        if not lead:
            scatter_drain(1 - seg)

        @pl.when(p + 1 < NP)
        def _():
            idx_start(p + 1, 1 - seg)

        for k in range(PH):
            pltpu.async_copy(ones_v, acc.at[dst_ring[seg].at[k]], ssem,
                             add=True)

    phase(0, 0, lead=True)
    phase(1, 1)

    def gbody(sp, _):
        phase(2 * sp + 2, 0)
        phase(2 * sp + 3, 1)
        return 0
    lax.fori_loop(0, (NP - 2) // 2, gbody, 0)
    scatter_drain(1)
    plsc.subcore_barrier()
    pltpu.sync_copy(acc.at[pl.ds(s * RPT, RPT)],
                    out_hbm.at[c, pl.ds(s * RPT, RPT)])


@functools.partial(
    pl.kernel,
    out_type=jax.ShapeDtypeStruct((NC, ACC, D), jnp.float32),
    mesh=_mesh,
    scratch_types=[
        pltpu.VMEM((PH, CH), jnp.int32),
        pltpu.VMEM((PH, CH), jnp.int32),
        pltpu.VMEM((PH, CH), jnp.int32),
        pltpu.VMEM((PH, CH), jnp.int32),
        pltpu.VMEM((CH, D), jnp.float32),
        pltpu.VMEM((CH, D), jnp.float32),
        pltpu.VMEM((16, D), jnp.float32),
        pltpu.VMEM_SHARED((ACC, D), jnp.float32),
        pltpu.SemaphoreType.DMA,
        pltpu.SemaphoreType.DMA,
        pltpu.SemaphoreType.DMA,
        pltpu.SemaphoreType.DMA,
        pltpu.SemaphoreType.DMA,
    ],
)
def _agg_sc(y_hbm, src_hbm, dst_hbm, out_hbm, sr0, sr1, dr0, dr1, buf0, buf1,
            zeros_v, acc, isem, gs0, gs1, ss0, ss1):
    gsems = (gs0, gs1)
    ssems = (ss0, ss1)
    bufs = (buf0, buf1)
    src_ring = (sr0, sr1)
    dst_ring = (dr0, dr1)
    c = lax.axis_index("c")
    s = lax.axis_index("s")
    wid = c * NS + s

    def idx_start(p, seg):
        pltpu.make_async_copy(src_hbm.at[wid, pl.ds(p * PH, PH)],
                              src_ring[seg], isem).start()
        pltpu.make_async_copy(dst_hbm.at[wid, pl.ds(p * PH, PH)],
                              dst_ring[seg], isem).start()

    def idx_wait(seg):
        pltpu.make_async_copy(src_hbm.at[wid, pl.ds(0, PH)],
                              src_ring[seg], isem).wait()
        pltpu.make_async_copy(dst_hbm.at[wid, pl.ds(0, PH)],
                              dst_ring[seg], isem).wait()

    idx_start(0, 0)
    for i in range(16):
        for j in range(D // 16):
            zeros_v[i, pl.ds(j * 16, 16)] = jnp.zeros((16,), jnp.float32)

    def zbody(i, _):
        pltpu.make_async_copy(zeros_v, acc.at[pl.ds(s * RPT + i * 16, 16)],
                              ss0).start()
        return 0
    lax.fori_loop(0, RPT // 16, zbody, 0)

    def zdrain(i, _):
        pltpu.make_async_copy(zeros_v, acc.at[pl.ds(s * RPT, 16)],
                              ss0).wait()
        return 0
    lax.fori_loop(0, RPT // 16, zdrain, 0)
    plsc.subcore_barrier()

    def gather_start(seg, k, b):
        pltpu.make_async_copy(y_hbm.at[src_ring[seg].at[k]], bufs[b],
                              gsems[b]).start()

    def gather_wait(seg, b):
        pltpu.make_async_copy(y_hbm.at[src_ring[seg].at[0]], bufs[b],
                              gsems[b]).wait()

    def scatter_start(seg, k, b):
        pltpu.async_copy(bufs[b], acc.at[dst_ring[seg].at[k]], ssems[b],
                         add=True)

    def scatter_wait(seg, b):
        pltpu.make_async_copy(bufs[b], acc.at[dst_ring[seg].at[0]],
                              ssems[b]).wait()

    def phase(p, seg, lead=False):
        idx_wait(seg)

        @pl.when(p + 1 < NP)
        def _():
            idx_start(p + 1, 1 - seg)

        gather_start(seg, 0, 0)
        for k in range(PH):
            b = k % 2
            if not (lead and k == 0):
                scatter_wait(seg, 1 - b)
            if k + 1 < PH:
                gather_start(seg, k + 1, 1 - b)
            gather_wait(seg, b)
            scatter_start(seg, k, b)

    phase(0, 0, lead=True)
    phase(1, 1)

    def gbody(sp, _):
        phase(2 * sp + 2, 0)
        phase(2 * sp + 3, 1)
        return 0
    lax.fori_loop(0, (NP - 2) // 2, gbody, 0)
    scatter_wait(1, (PH - 1) % 2)
    plsc.subcore_barrier()
    pltpu.sync_copy(acc.at[pl.ds(s * RPT, RPT)],
                    out_hbm.at[c, pl.ds(s * RPT, RPT)])


def _dinv_body(d0, d1, o):
    o[...] = lax.rsqrt(d0[0][:, 0:1] + d1[0][:, 0:1] + 1.0)


_dinv_tc = pl.pallas_call(
    _dinv_body,
    out_shape=jax.ShapeDtypeStruct((N, 1), jnp.float32),
    grid=(N // RCH,),
    in_specs=[pl.BlockSpec((1, RCH, D), lambda i: (0, i, 0)),
              pl.BlockSpec((1, RCH, D), lambda i: (1, i, 0))],
    out_specs=pl.BlockSpec((RCH, 1), lambda i: (i, 0)),
)


def _mm_scale_body(x, w, dinv, y):
    y[...] = dinv[...] * jnp.dot(x[...], w[...],
                                 preferred_element_type=jnp.float32)


_mm_scale_tc = pl.pallas_call(
    _mm_scale_body,
    out_shape=jax.ShapeDtypeStruct((N, D), jnp.float32),
    grid=(N // RCH,),
    in_specs=[pl.BlockSpec((RCH, D), lambda i: (i, 0)),
              pl.BlockSpec((D, D), lambda i: (0, 0)),
              pl.BlockSpec((RCH, 1), lambda i: (i, 0))],
    out_specs=pl.BlockSpec((RCH, D), lambda i: (i, 0)),
)


def _comb_body(a0, a1, yp, dinv, b, w, yn):
    h = jnp.maximum(dinv[...] * (a0[0] + a1[0] + yp[...]) + b[...], 0.0)
    yn[...] = dinv[...] * jnp.dot(h, w[...], preferred_element_type=jnp.float32)


_comb_tc = pl.pallas_call(
    _comb_body,
    out_shape=jax.ShapeDtypeStruct((N, D), jnp.float32),
    grid=(N // RCH,),
    in_specs=[pl.BlockSpec((1, RCH, D), lambda i: (0, i, 0)),
              pl.BlockSpec((1, RCH, D), lambda i: (1, i, 0)),
              pl.BlockSpec((RCH, D), lambda i: (i, 0)),
              pl.BlockSpec((RCH, 1), lambda i: (i, 0)),
              pl.BlockSpec((1, D), lambda i: (0, 0)),
              pl.BlockSpec((D, D), lambda i: (0, 0))],
    out_specs=pl.BlockSpec((RCH, D), lambda i: (i, 0)),
)


def _pool_body(a0, a1, yp, dinv, b, batch, out, sums, cnts):
    i = pl.program_id(0)

    @pl.when(i == 0)
    def _():
        sums[...] = jnp.zeros_like(sums)
        cnts[...] = jnp.zeros_like(cnts)

    h = jnp.maximum(dinv[...] * (a0[0] + a1[0] + yp[...]) + b[...], 0.0)
    bt = batch[...].reshape(1, RCH)
    gids = lax.broadcasted_iota(jnp.int32, (G, RCH), 0)
    onehot_t = (gids == bt).astype(jnp.float32)
    sums[...] += jnp.dot(onehot_t, h, preferred_element_type=jnp.float32)
    cnts[...] += jnp.dot(onehot_t, jnp.ones((RCH, D), jnp.float32),
                         preferred_element_type=jnp.float32)

    @pl.when(i == pl.num_programs(0) - 1)
    def _():
        out[...] = sums[...] / jnp.maximum(cnts[...], 1.0)


_pool_tc = pl.pallas_call(
    _pool_body,
    out_shape=jax.ShapeDtypeStruct((G, D), jnp.float32),
    grid=(N // RCH,),
    in_specs=[pl.BlockSpec((1, RCH, D), lambda i: (0, i, 0)),
              pl.BlockSpec((1, RCH, D), lambda i: (1, i, 0)),
              pl.BlockSpec((RCH, D), lambda i: (i, 0)),
              pl.BlockSpec((RCH, 1), lambda i: (i, 0)),
              pl.BlockSpec((1, D), lambda i: (0, 0)),
              pl.BlockSpec((1, 1, RCH), lambda i: (i, 0, 0))],
    out_specs=pl.BlockSpec((G, D), lambda i: (0, 0)),
    scratch_shapes=[pltpu.VMEM((G, D), jnp.float32),
                    pltpu.VMEM((G, D), jnp.float32)],
)


@jax.jit
def kernel(x, edge_index, batch, W1, b1, W2, b2, W3, b3, W4, b4):
    src = edge_index[0].astype(jnp.int32)
    dst = edge_index[1].astype(jnp.int32)
    npad = E_PAD - E
    pad_src = (jnp.arange(npad, dtype=jnp.int32) * 64) % N
    pad_dst = N + (jnp.arange(npad, dtype=jnp.int32) % (ACC - N))
    srcp = jnp.concatenate([src, pad_src])
    dstp = jnp.concatenate([dst, pad_dst])
    srcp = srcp.reshape(NW, CPW, CH)
    dstp = dstp.reshape(NW, CPW, CH)
    batch2 = batch.astype(jnp.int32).reshape(N // RCH, 1, RCH)

    degp = _deg_sc(dstp)
    dinv = _dinv_tc(degp, degp)

    b1r = b1.reshape(1, D)
    b2r = b2.reshape(1, D)
    b3r = b3.reshape(1, D)
    b4r = b4.reshape(1, D)

    y = _mm_scale_tc(x, W1, dinv)
    for (br, Wn) in ((b1r, W2), (b2r, W3), (b3r, W4)):
        ag = _agg_sc(y, srcp, dstp)
        y = _comb_tc(ag, ag, y, dinv, br, Wn)
    ag = _agg_sc(y, srcp, dstp)
    out = _pool_tc(ag, ag, y, dinv, b4r, batch2)
    return out
